# baseline (speedup 1.0000x reference)
import functools

import jax
import jax.numpy as jnp
from jax import lax
from jax.experimental import pallas as pl
from jax.experimental.pallas import tpu as pltpu
from jax.experimental.pallas import tpu_sc as plsc

N = 50000
C = 128
G = 256
NOFF = 26
CAP = 512
ROWS = NOFF * CAP
R_CAP = 6
U_PC = 2048
ZR = CAP - 1
BIG = 2 ** 30

NSC = 2
NSUB = 16
NW = NSC * NSUB
RPW = ROWS // NW
GCH = 104
UPS = U_PC // NSUB
N_PAD = 50176
ZPW = N_PAD // NW
ZCH = 224
HALF = N_PAD // 2
BLK = 1024


def _wfold_body(wc_ref, wl_ref, out_ref):
    out_ref[0] = jnp.dot(wc_ref[0], wl_ref[...],
                         preferred_element_type=jnp.float32)


_wfold = pl.pallas_call(
    _wfold_body,
    grid=(27,),
    in_specs=[
        pl.BlockSpec((1, C, C), lambda k: (k, 0, 0)),
        pl.BlockSpec((C, C), lambda k: (0, 0)),
    ],
    out_specs=pl.BlockSpec((1, C, C), lambda k: (k, 0, 0)),
    out_shape=jax.ShapeDtypeStruct((27, C, C), jnp.float32),
)


def _corr_body(gc_ref, valid_ref, wf_ref, out_ref):
    g = gc_ref[0] * valid_ref[0]
    out_ref[0] = jnp.dot(g, wf_ref[0], preferred_element_type=jnp.float32)


_corr_mm = pl.pallas_call(
    _corr_body,
    grid=(NOFF,),
    in_specs=[
        pl.BlockSpec((1, CAP, C), lambda k: (k, 0, 0)),
        pl.BlockSpec((1, CAP, 1), lambda k: (k, 0, 0)),
        pl.BlockSpec((1, C, C), lambda k: (k + (k >= 13).astype(jnp.int32), 0, 0)),
    ],
    out_specs=pl.BlockSpec((1, CAP, C), lambda k: (k, 0, 0)),
    out_shape=jax.ShapeDtypeStruct((NOFF, CAP, C), jnp.float32),
)


def _main_body(f_ref, c_ref, wf_ref, bf_ref, g_ref, b_ref, out_ref):
    h = jnp.dot(f_ref[...], wf_ref[0], preferred_element_type=jnp.float32)
    h = h + c_ref[...] + bf_ref[...]
    mu = jnp.mean(h, axis=1, keepdims=True)
    hc = h - mu
    var = jnp.mean(hc * hc, axis=1, keepdims=True)
    out_ref[...] = hc * lax.rsqrt(var + 1e-5) * g_ref[...] + b_ref[...]


_main = pl.pallas_call(
    _main_body,
    grid=(pl.cdiv(N, BLK),),
    in_specs=[
        pl.BlockSpec((BLK, C), lambda i: (i, 0)),
        pl.BlockSpec((BLK, C), lambda i: (i, 0)),
        pl.BlockSpec((1, C, C), lambda i: (13, 0, 0)),
        pl.BlockSpec((1, C), lambda i: (0, 0)),
        pl.BlockSpec((1, C), lambda i: (0, 0)),
        pl.BlockSpec((1, C), lambda i: (0, 0)),
    ],
    out_specs=pl.BlockSpec((BLK, C), lambda i: (i, 0)),
    out_shape=jax.ShapeDtypeStruct((N, C), jnp.float32),
)


@functools.lru_cache(maxsize=None)
def _sc_kernels():
    mesh = plsc.VectorSubcoreMesh(core_axis_name="c", subcore_axis_name="s",
                                  num_cores=NSC, num_subcores=NSUB)

    @functools.partial(
        pl.kernel,
        out_type=jax.ShapeDtypeStruct((ROWS, C), jnp.float32),
        mesh=mesh,
        scratch_types=[
            pltpu.VMEM((GCH,), jnp.int32),
            pltpu.VMEM((GCH, C), jnp.float32),
            pltpu.SemaphoreType.DMA,
        ],
    )
    def sc_gather(f_hbm, src_hbm, out_hbm, idx_v, rows_v, sem):
        wid = lax.axis_index("c") * NSUB + lax.axis_index("s")
        base = wid * RPW
        for ch in range(RPW // GCH):
            start = base + ch * GCH
            pltpu.sync_copy(src_hbm.at[pl.ds(start, GCH)], idx_v)
            pltpu.async_copy(f_hbm.at[idx_v], rows_v, sem).wait()
            pltpu.sync_copy(rows_v, out_hbm.at[pl.ds(start, GCH)])

    @functools.partial(
        pl.kernel,
        out_type=jax.ShapeDtypeStruct((N_PAD, C), jnp.float32),
        mesh=mesh,
        scratch_types=[
            pltpu.VMEM((ZCH, C), jnp.float32),
            pltpu.VMEM((UPS,), jnp.int32),
            pltpu.VMEM((UPS,), jnp.int32),
            pltpu.VMEM((UPS, C), jnp.float32),
            pltpu.VMEM((UPS, C), jnp.float32),
            pltpu.SemaphoreType.DMA,
        ],
    )
    def sc_scatter(corr_hbm, src2_hbm, udst_hbm, out_hbm,
                   zbuf, idx_v, udst_v, acc, tmp, sem):
        cid = lax.axis_index("c")
        sid = lax.axis_index("s")
        wid = cid * NSUB + sid

        zero16 = jnp.zeros((16,), jnp.float32)

        def _zrow(i, _):
            for v in range(C // 16):
                zbuf[i, pl.ds(v * 16, 16)] = zero16
            return 0

        lax.fori_loop(0, ZCH, _zrow, 0)
        zbase = wid * ZPW
        for j in range(ZPW // ZCH):
            pltpu.sync_copy(zbuf, out_hbm.at[pl.ds(zbase + j * ZCH, ZCH)])

        plsc.subcore_barrier()

        ubase = sid * UPS
        pltpu.sync_copy(udst_hbm.at[cid, pl.ds(ubase, UPS)], udst_v)
        pltpu.sync_copy(src2_hbm.at[cid, 0, pl.ds(ubase, UPS)], idx_v)
        pltpu.async_copy(corr_hbm.at[idx_v], acc, sem).wait()
        for r in range(1, R_CAP):
            pltpu.sync_copy(src2_hbm.at[cid, r, pl.ds(ubase, UPS)], idx_v)
            pltpu.async_copy(corr_hbm.at[idx_v], tmp, sem).wait()

            def _arow(i, _):
                for v in range(C // 16):
                    sl = pl.ds(v * 16, 16)
                    acc[i, sl] = acc[i, sl] + tmp[i, sl]
                return 0

            lax.fori_loop(0, UPS, _arow, 0)

        pltpu.async_copy(acc, out_hbm.at[udst_v], sem).wait()

    return sc_gather, sc_scatter


def _neighbor_tables(coords):
    ci = coords.astype(jnp.int32)
    keys = ci[:, 0] * (G * G) + ci[:, 1] * G + ci[:, 2]
    order = jnp.argsort(keys)
    skeys = keys[order]
    offs = jnp.array([[dx, dy, dz]
                      for dx in (-1, 0, 1)
                      for dy in (-1, 0, 1)
                      for dz in (-1, 0, 1)], jnp.int32)
    nc = ci[None, :, :] + offs[:, None, :]
    inb = jnp.all((nc >= 0) & (nc < G), axis=2)
    nkey = nc[..., 0] * (G * G) + nc[..., 1] * G + nc[..., 2]
    pos = jnp.clip(jnp.searchsorted(skeys, nkey.reshape(-1)).reshape(27, N),
                   0, N - 1)
    found = inb & (skeys[pos] == nkey)
    src = order[pos]
    return found, src


def _build_fast_path_indices(found, src):
    nc_ids = jnp.array([k for k in range(27) if k != 13], jnp.int32)
    fnd = found[nc_ids]
    srcn = src[nc_ids]
    posi = jnp.cumsum(fnd.astype(jnp.int32), axis=1) - 1
    cnt = posi[:, -1] + 1
    overflow = jnp.any(cnt >= CAP)

    row = jnp.broadcast_to(jnp.arange(NOFF, dtype=jnp.int32)[:, None], (NOFF, N))
    col = jnp.where(fnd & (posi < CAP), posi, CAP)
    src_pad = jnp.zeros((NOFF, CAP + 1), jnp.int32).at[row, col].set(
        srcn, mode="drop")[:, :CAP]
    valid = jnp.zeros((NOFF, CAP + 1), jnp.float32).at[row, col].set(
        1.0, mode="drop")[:, :CAP]
    dstn = jnp.broadcast_to(jnp.arange(N, dtype=jnp.int32)[None, :], (NOFF, N))
    dst_pad = jnp.full((NOFF, CAP + 1), BIG, jnp.int32).at[row, col].set(
        dstn, mode="drop")[:, :CAP]

    P = ROWS
    dst_flat = dst_pad.reshape(P)
    sort_idx = jnp.argsort(dst_flat).astype(jnp.int32)
    sdst = dst_flat[sort_idx]
    valid_h = sdst < BIG
    flag = jnp.concatenate([jnp.ones((1,), bool),
                            sdst[1:] != sdst[:-1]]) & valid_h
    uid = jnp.cumsum(flag.astype(jnp.int32)) - 1
    ar = jnp.arange(P, dtype=jnp.int32)
    runstart = lax.cummax(jnp.where(flag, ar, 0))
    rpos = ar - runstart
    maxrun = jnp.max(jnp.where(valid_h, rpos, 0)) + 1
    overflow |= maxrun > R_CAP

    sc_h = (sdst >= HALF).astype(jnp.int32)
    u0 = jnp.sum((flag & (sdst < HALF)).astype(jnp.int32))
    u_total = jnp.sum(flag.astype(jnp.int32))
    overflow |= (u0 > U_PC) | ((u_total - u0) > U_PC)
    ul = uid - sc_h * u0

    hashit = jnp.zeros((HALF,), bool).at[
        jnp.where(valid_h & (sdst < HALF), sdst, HALF)].set(True, mode="drop")
    hole0 = jnp.argmin(hashit).astype(jnp.int32)

    i0 = jnp.where(valid_h, sc_h, NSC)
    i1 = jnp.where(rpos < R_CAP, rpos, R_CAP)
    i2 = jnp.where(ul < U_PC, ul, U_PC)
    udst_pk = jnp.stack([jnp.zeros((U_PC,), jnp.int32) + hole0,
                         jnp.full((U_PC,), N, jnp.int32)])
    udst_pk = udst_pk.at[jnp.where(flag, sc_h, NSC), i2].set(sdst, mode="drop")
    src2_pk = jnp.full((NSC, R_CAP, U_PC), ZR, jnp.int32)
    src2_pk = src2_pk.at[i0, i1, i2].set(sort_idx, mode="drop")

    return (src_pad.reshape(ROWS), valid.reshape(NOFF, CAP, 1),
            src2_pk, udst_pk, overflow)


def kernel(features, coords, W_conv, b_conv, W_lin, b_lin, gamma, beta):
    found, src = _neighbor_tables(coords)
    src_pad, valid3, src2_pk, udst_pk, overflow = _build_fast_path_indices(
        found, src)

    Wf = _wfold(W_conv, W_lin)
    bf = (b_conv @ W_lin + b_lin).reshape(1, C)
    g2 = gamma.reshape(1, C)
    b2 = beta.reshape(1, C)

    def _fast(_):
        sc_gather, sc_scatter = _sc_kernels()
        gc = sc_gather(features, src_pad)
        corr = _corr_mm(gc.reshape(NOFF, CAP, C), valid3, Wf)
        cbuf = sc_scatter(corr.reshape(ROWS, C), src2_pk, udst_pk)
        return _main(features, cbuf, Wf, bf, g2, b2)

    def _dense(_):
        gath = features[src] * found[..., None].astype(features.dtype)
        h = jnp.einsum("knc,kco->no", gath, W_conv) + b_conv
        h = h @ W_lin + b_lin
        mu = jnp.mean(h, axis=-1, keepdims=True)
        var = jnp.var(h, axis=-1, keepdims=True)
        return (h - mu) / jnp.sqrt(var + 1e-5) * gamma + beta

    return lax.cond(overflow, _dense, _fast, 0)


# speedup vs baseline: 1.0971x; 1.0971x over previous
import functools

import jax
import jax.numpy as jnp
from jax import lax
from jax.experimental import pallas as pl
from jax.experimental.pallas import tpu as pltpu
from jax.experimental.pallas import tpu_sc as plsc

N = 50000
C = 128
G = 256
NOFF = 26
CAP = 512
ROWS = NOFF * CAP
R_CAP = 6
U_PC = 2048
ZR = CAP - 1
BIG = 2 ** 30

NSC = 2
NSUB = 16
NW = NSC * NSUB
RPW = ROWS // NW
GCH = 104
UPS = U_PC // NSUB
N_PAD = 50176
ZPW = N_PAD // NW
ZCH = 224
HALF = N_PAD // 2
BLK = 1024


def _wfold_body(wc_ref, wl_ref, out_ref):
    out_ref[0] = jnp.dot(wc_ref[0], wl_ref[...],
                         preferred_element_type=jnp.float32)


_wfold = pl.pallas_call(
    _wfold_body,
    grid=(27,),
    in_specs=[
        pl.BlockSpec((1, C, C), lambda k: (k, 0, 0)),
        pl.BlockSpec((C, C), lambda k: (0, 0)),
    ],
    out_specs=pl.BlockSpec((1, C, C), lambda k: (k, 0, 0)),
    out_shape=jax.ShapeDtypeStruct((27, C, C), jnp.float32),
)


def _corr_body(gc_ref, valid_ref, wf_ref, out_ref):
    g = gc_ref[0] * valid_ref[0]
    out_ref[0] = jnp.dot(g, wf_ref[0], preferred_element_type=jnp.float32)


_corr_mm = pl.pallas_call(
    _corr_body,
    grid=(NOFF,),
    in_specs=[
        pl.BlockSpec((1, CAP, C), lambda k: (k, 0, 0)),
        pl.BlockSpec((1, CAP, 1), lambda k: (k, 0, 0)),
        pl.BlockSpec((1, C, C), lambda k: (k + (k >= 13).astype(jnp.int32), 0, 0)),
    ],
    out_specs=pl.BlockSpec((1, CAP, C), lambda k: (k, 0, 0)),
    out_shape=jax.ShapeDtypeStruct((NOFF, CAP, C), jnp.float32),
)


def _main_body(f_ref, c_ref, wf_ref, bf_ref, g_ref, b_ref, out_ref):
    h = jnp.dot(f_ref[...], wf_ref[0], preferred_element_type=jnp.float32)
    h = h + c_ref[...] + bf_ref[...]
    mu = jnp.mean(h, axis=1, keepdims=True)
    hc = h - mu
    var = jnp.mean(hc * hc, axis=1, keepdims=True)
    out_ref[...] = hc * lax.rsqrt(var + 1e-5) * g_ref[...] + b_ref[...]


_main = pl.pallas_call(
    _main_body,
    grid=(pl.cdiv(N, BLK),),
    in_specs=[
        pl.BlockSpec((BLK, C), lambda i: (i, 0)),
        pl.BlockSpec((BLK, C), lambda i: (i, 0)),
        pl.BlockSpec((1, C, C), lambda i: (13, 0, 0)),
        pl.BlockSpec((1, C), lambda i: (0, 0)),
        pl.BlockSpec((1, C), lambda i: (0, 0)),
        pl.BlockSpec((1, C), lambda i: (0, 0)),
    ],
    out_specs=pl.BlockSpec((BLK, C), lambda i: (i, 0)),
    out_shape=jax.ShapeDtypeStruct((N, C), jnp.float32),
)


@functools.lru_cache(maxsize=None)
def _sc_kernels():
    mesh = plsc.VectorSubcoreMesh(core_axis_name="c", subcore_axis_name="s",
                                  num_cores=NSC, num_subcores=NSUB)

    @functools.partial(
        pl.kernel,
        out_type=jax.ShapeDtypeStruct((ROWS, C), jnp.float32),
        mesh=mesh,
        scratch_types=[
            pltpu.VMEM((GCH,), jnp.int32),
            pltpu.VMEM((GCH, C), jnp.float32),
            pltpu.SemaphoreType.DMA,
        ],
    )
    def sc_gather(f_hbm, src_hbm, out_hbm, idx_v, rows_v, sem):
        wid = lax.axis_index("c") * NSUB + lax.axis_index("s")
        base = wid * RPW
        for ch in range(RPW // GCH):
            start = base + ch * GCH
            pltpu.sync_copy(src_hbm.at[pl.ds(start, GCH)], idx_v)
            pltpu.async_copy(f_hbm.at[idx_v], rows_v, sem).wait()
            pltpu.sync_copy(rows_v, out_hbm.at[pl.ds(start, GCH)])

    @functools.partial(
        pl.kernel,
        out_type=jax.ShapeDtypeStruct((N_PAD, C), jnp.float32),
        mesh=mesh,
        scratch_types=[
            pltpu.VMEM((ZCH, C), jnp.float32),
            pltpu.VMEM((UPS,), jnp.int32),
            pltpu.VMEM((UPS,), jnp.int32),
            pltpu.VMEM((UPS, C), jnp.float32),
            pltpu.VMEM((UPS, C), jnp.float32),
            pltpu.SemaphoreType.DMA,
        ],
    )
    def sc_scatter(corr_hbm, src2_hbm, udst_hbm, out_hbm,
                   zbuf, idx_v, udst_v, acc, tmp, sem):
        cid = lax.axis_index("c")
        sid = lax.axis_index("s")
        wid = cid * NSUB + sid

        zero16 = jnp.zeros((16,), jnp.float32)

        def _zrow(i, _):
            for v in range(C // 16):
                zbuf[i, pl.ds(v * 16, 16)] = zero16
            return 0

        lax.fori_loop(0, ZCH, _zrow, 0)
        zbase = wid * ZPW
        for j in range(ZPW // ZCH):
            pltpu.sync_copy(zbuf, out_hbm.at[pl.ds(zbase + j * ZCH, ZCH)])

        plsc.subcore_barrier()

        ubase = sid * UPS
        pltpu.sync_copy(udst_hbm.at[cid, pl.ds(ubase, UPS)], udst_v)
        pltpu.sync_copy(src2_hbm.at[cid, 0, pl.ds(ubase, UPS)], idx_v)
        pltpu.async_copy(corr_hbm.at[idx_v], acc, sem).wait()
        for r in range(1, R_CAP):
            pltpu.sync_copy(src2_hbm.at[cid, r, pl.ds(ubase, UPS)], idx_v)
            pltpu.async_copy(corr_hbm.at[idx_v], tmp, sem).wait()

            def _arow(i, _):
                for v in range(C // 16):
                    sl = pl.ds(v * 16, 16)
                    acc[i, sl] = acc[i, sl] + tmp[i, sl]
                return 0

            lax.fori_loop(0, UPS, _arow, 0)

        pltpu.async_copy(acc, out_hbm.at[udst_v], sem).wait()

    return sc_gather, sc_scatter


def _neighbor_tables(coords):
    ci = coords.astype(jnp.int32)
    keys = ci[:, 0] * (G * G) + ci[:, 1] * G + ci[:, 2]
    order = jnp.argsort(keys)
    skeys = keys[order]
    offs = jnp.array([[dx, dy, dz]
                      for dx in (-1, 0, 1)
                      for dy in (-1, 0, 1)
                      for dz in (-1, 0, 1)], jnp.int32)
    nc = ci[None, :, :] + offs[:, None, :]
    inb = jnp.all((nc >= 0) & (nc < G), axis=2)
    nkey = nc[..., 0] * (G * G) + nc[..., 1] * G + nc[..., 2]
    pos = jnp.clip(jnp.searchsorted(skeys, nkey.reshape(-1)).reshape(27, N),
                   0, N - 1)
    found = inb & (skeys[pos] == nkey)
    src = order[pos]
    return found, src


def _build_fast_path_indices(found, src):
    nc_ids = jnp.array([k for k in range(27) if k != 13], jnp.int32)
    fnd = found[nc_ids]
    srcn = src[nc_ids]
    cnt = jnp.sum(fnd.astype(jnp.int32), axis=1)
    overflow = jnp.any(cnt >= CAP)

    valid, perm = lax.top_k(fnd.astype(jnp.float32), CAP)
    perm = perm.astype(jnp.int32)
    src_pad = jnp.take_along_axis(srcn, perm, axis=1)
    dst_pad = jnp.where(valid > 0, perm, BIG)

    P = ROWS
    dst_flat = dst_pad.reshape(P)
    sort_idx = jnp.argsort(dst_flat).astype(jnp.int32)
    sdst = dst_flat[sort_idx]
    valid_h = sdst < BIG
    flag = jnp.concatenate([jnp.ones((1,), bool),
                            sdst[1:] != sdst[:-1]]) & valid_h
    ar = jnp.arange(P, dtype=jnp.int32)
    runstart = lax.cummax(jnp.where(flag, ar, 0))
    rpos = ar - runstart
    maxrun = jnp.max(jnp.where(valid_h, rpos, 0)) + 1
    overflow |= maxrun > R_CAP

    u0 = jnp.sum((flag & (sdst < HALF)).astype(jnp.int32))
    u_total = jnp.sum(flag.astype(jnp.int32))
    overflow |= (u0 > U_PC) | ((u_total - u0) > U_PC)

    ustart = jnp.argsort(jnp.logical_not(flag), stable=True).astype(jnp.int32)

    gaps = (sdst[1:] > sdst[:-1] + 1) & valid_h[:-1] & (sdst[:-1] + 1 < HALF)
    jj = jnp.argmax(gaps)
    hole0 = jnp.where(sdst[0] != 0, 0, sdst[jj] + 1).astype(jnp.int32)

    sc_ix = jnp.arange(NSC, dtype=jnp.int32)[:, None, None]
    u_ix = jnp.arange(U_PC, dtype=jnp.int32)[None, None, :]
    r_ix = jnp.arange(R_CAP, dtype=jnp.int32)[None, :, None]
    guid = sc_ix * u0 + u_ix
    uvalid = jnp.where(sc_ix == 0, u_ix < u0, guid < u_total)
    st = ustart[jnp.minimum(guid, P - 1)]
    q = st + r_ix
    qc = jnp.minimum(q, P - 1)
    entry_ok = uvalid & (q < P) & (sdst[qc] == sdst[st])
    src2_pk = jnp.where(entry_ok, sort_idx[qc], ZR)
    hole = jnp.stack([hole0, jnp.int32(N)])[:, None]
    udst_pk = jnp.where(uvalid[:, 0, :], sdst[st[:, 0, :]], hole)

    return (src_pad.reshape(ROWS), valid.reshape(NOFF, CAP, 1),
            src2_pk, udst_pk, overflow)


def kernel(features, coords, W_conv, b_conv, W_lin, b_lin, gamma, beta):
    found, src = _neighbor_tables(coords)
    src_pad, valid3, src2_pk, udst_pk, overflow = _build_fast_path_indices(
        found, src)

    Wf = _wfold(W_conv, W_lin)
    bf = (b_conv @ W_lin + b_lin).reshape(1, C)
    g2 = gamma.reshape(1, C)
    b2 = beta.reshape(1, C)

    def _fast(_):
        sc_gather, sc_scatter = _sc_kernels()
        gc = sc_gather(features, src_pad)
        corr = _corr_mm(gc.reshape(NOFF, CAP, C), valid3, Wf)
        cbuf = sc_scatter(corr.reshape(ROWS, C), src2_pk, udst_pk)
        return _main(features, cbuf, Wf, bf, g2, b2)

    def _dense(_):
        gath = features[src] * found[..., None].astype(features.dtype)
        h = jnp.einsum("knc,kco->no", gath, W_conv) + b_conv
        h = h @ W_lin + b_lin
        mu = jnp.mean(h, axis=-1, keepdims=True)
        var = jnp.var(h, axis=-1, keepdims=True)
        return (h - mu) / jnp.sqrt(var + 1e-5) * gamma + beta

    return lax.cond(overflow, _dense, _fast, 0)


# speedup vs baseline: 79.4030x; 72.3779x over previous
import functools

import jax
import jax.numpy as jnp
from jax import lax
from jax.experimental import pallas as pl
from jax.experimental.pallas import tpu as pltpu
from jax.experimental.pallas import tpu_sc as plsc

N = 50000
C = 128
G = 256
NOFF = 26
NSC = 2
NSUB = 16
NW = NSC * NSUB
CAPW = 32
CAP = NW * CAPW
ROWS = NOFF * CAP
R_CAP = 6
U_PC = 2048
ZR = CAPW - 1
BIG = 2 ** 30
PADKEY = -(2 ** 20)

RPW = ROWS // NW
GCH = 104
UPS = U_PC // NSUB
N_PAD = 50176
KPW = N_PAD // NW
ZPW = N_PAD // NW
ZCH = 224
HALF = N_PAD // 2
BLK = 1024
SRB = NOFF * CAPW
SRB_PAD = 896
SKP = 50048

OFFS26 = [(dx, dy, dz)
          for dx in (-1, 0, 1) for dy in (-1, 0, 1) for dz in (-1, 0, 1)
          if not (dx == 0 and dy == 0 and dz == 0)]


def _wfold_body(wc_ref, wl_ref, out_ref):
    out_ref[0] = jnp.dot(wc_ref[0], wl_ref[...],
                         preferred_element_type=jnp.float32)


_wfold = pl.pallas_call(
    _wfold_body,
    grid=(27,),
    in_specs=[
        pl.BlockSpec((1, C, C), lambda k: (k, 0, 0)),
        pl.BlockSpec((C, C), lambda k: (0, 0)),
    ],
    out_specs=pl.BlockSpec((1, C, C), lambda k: (k, 0, 0)),
    out_shape=jax.ShapeDtypeStruct((27, C, C), jnp.float32),
)


def _corr_body(gc_ref, valid_ref, wf_ref, out_ref):
    g = gc_ref[0] * valid_ref[0]
    out_ref[0] = jnp.dot(g, wf_ref[0], preferred_element_type=jnp.float32)


_corr_mm = pl.pallas_call(
    _corr_body,
    grid=(NOFF,),
    in_specs=[
        pl.BlockSpec((1, CAP, C), lambda k: (k, 0, 0)),
        pl.BlockSpec((1, CAP, 1), lambda k: (k, 0, 0)),
        pl.BlockSpec((1, C, C), lambda k: (k + (k >= 13).astype(jnp.int32), 0, 0)),
    ],
    out_specs=pl.BlockSpec((1, CAP, C), lambda k: (k, 0, 0)),
    out_shape=jax.ShapeDtypeStruct((NOFF, CAP, C), jnp.float32),
)


def _main_body(f_ref, c_ref, wf_ref, bf_ref, g_ref, b_ref, out_ref):
    h = jnp.dot(f_ref[...], wf_ref[0], preferred_element_type=jnp.float32)
    h = h + c_ref[...] + bf_ref[...]
    mu = jnp.mean(h, axis=1, keepdims=True)
    hc = h - mu
    var = jnp.mean(hc * hc, axis=1, keepdims=True)
    out_ref[...] = hc * lax.rsqrt(var + 1e-5) * g_ref[...] + b_ref[...]


_main = pl.pallas_call(
    _main_body,
    grid=(pl.cdiv(N, BLK),),
    in_specs=[
        pl.BlockSpec((BLK, C), lambda i: (i, 0)),
        pl.BlockSpec((BLK, C), lambda i: (i, 0)),
        pl.BlockSpec((1, C, C), lambda i: (13, 0, 0)),
        pl.BlockSpec((1, C), lambda i: (0, 0)),
        pl.BlockSpec((1, C), lambda i: (0, 0)),
        pl.BlockSpec((1, C), lambda i: (0, 0)),
    ],
    out_specs=pl.BlockSpec((BLK, C), lambda i: (i, 0)),
    out_shape=jax.ShapeDtypeStruct((N, C), jnp.float32),
)


@functools.lru_cache(maxsize=None)
def _sc_kernels():
    mesh = plsc.VectorSubcoreMesh(core_axis_name="c", subcore_axis_name="s",
                                  num_cores=NSC, num_subcores=NSUB)

    @functools.partial(
        pl.kernel,
        out_type=(jax.ShapeDtypeStruct((NW * SRB,), jnp.int32),
                  jax.ShapeDtypeStruct((NW * SRB,), jnp.int32),
                  jax.ShapeDtypeStruct((NW * 32,), jnp.int32)),
        mesh=mesh,
        compiler_params=pltpu.CompilerParams(needs_layout_passes=False),
        scratch_types=[
            pltpu.VMEM((KPW,), jnp.int32),
            pltpu.VMEM((SKP,), jnp.int32),
            pltpu.VMEM((SKP,), jnp.int32),
            pltpu.VMEM((SRB_PAD,), jnp.int32),
            pltpu.VMEM((SRB_PAD,), jnp.int32),
            pltpu.VMEM((32,), jnp.int32),
        ],
    )
    def sc_search(keys_hbm, skeys_hbm, order_hbm,
                  src_hbm, dst_hbm, cnt_hbm,
                  mykeys, skeys_v, order_v, srcb, dstb, cntb):
        wid = lax.axis_index("c") * NSUB + lax.axis_index("s")
        pltpu.sync_copy(keys_hbm.at[pl.ds(wid * KPW, KPW)], mykeys)
        pltpu.sync_copy(skeys_hbm, skeys_v)
        pltpu.sync_copy(order_hbm, order_v)

        zero16 = jnp.zeros((16,), jnp.int32)
        big16 = jnp.full((16,), BIG, jnp.int32)
        for i in range(SRB // 16):
            srcb[pl.ds(i * 16, 16)] = zero16
            dstb[pl.ds(i * 16, 16)] = big16
        iota16 = lax.iota(jnp.int32, 16)

        def row_step(t, cnts):
            key = mykeys[pl.ds(t * 16, 16)]
            dstv = wid * KPW + t * 16 + iota16
            x = key >> 16
            y = (key >> 8) & 255
            z = key & 255
            new_cnts = []
            for j, (dx, dy, dz) in enumerate(OFFS26):
                xx = x + dx
                yy = y + dy
                zz = z + dz
                m = ((xx >= 0) & (xx < G) & (yy >= 0) & (yy < G)
                     & (zz >= 0) & (zz < G))
                nkey = key + (dx * G * G + dy * G + dz)

                def bstep(s, lohi):
                    lo, hi = lohi
                    mid = (lo + hi) >> 1
                    sk = plsc.load_gather(skeys_v, [mid])
                    less = sk < nkey
                    return (jnp.where(less, mid + 1, lo),
                            jnp.where(less, hi, mid))

                lo, _hi = lax.fori_loop(
                    0, 16, bstep,
                    (jnp.zeros((16,), jnp.int32),
                     jnp.full((16,), N, jnp.int32)))
                posc = jnp.minimum(lo, N - 1)
                skp = plsc.load_gather(skeys_v, [posc])
                fnd = m & (lo < N) & (skp == nkey)
                srcv = plsc.load_gather(order_v, [posc])
                fi = fnd.astype(jnp.int32)
                csum = plsc.cumsum(fi)
                posb = j * CAPW + cnts[j] + csum - fi
                okw = fnd & (csum - fi + cnts[j] < CAPW)
                plsc.store_scatter(srcb, [posb], srcv, mask=okw)
                plsc.store_scatter(dstb, [posb], dstv, mask=okw)
                new_cnts.append(cnts[j] + jnp.sum(fi))
            return tuple(new_cnts)

        cnts_fin = lax.fori_loop(0, KPW // 16, row_step,
                                 tuple(jnp.int32(0) for _ in range(NOFF)))

        c0 = zero16
        c1 = zero16
        for k in range(16):
            c0 = jnp.where(iota16 == k, cnts_fin[k], c0)
        for k in range(16, NOFF):
            c1 = jnp.where(iota16 == (k - 16), cnts_fin[k], c1)
        cntb[pl.ds(0, 16)] = c0
        cntb[pl.ds(16, 16)] = c1

        pltpu.sync_copy(srcb.at[pl.ds(0, SRB)],
                        src_hbm.at[pl.ds(wid * SRB, SRB)])
        pltpu.sync_copy(dstb.at[pl.ds(0, SRB)],
                        dst_hbm.at[pl.ds(wid * SRB, SRB)])
        pltpu.sync_copy(cntb, cnt_hbm.at[pl.ds(wid * 32, 32)])

    @functools.partial(
        pl.kernel,
        out_type=jax.ShapeDtypeStruct((ROWS, C), jnp.float32),
        mesh=mesh,
        scratch_types=[
            pltpu.VMEM((GCH,), jnp.int32),
            pltpu.VMEM((GCH, C), jnp.float32),
            pltpu.SemaphoreType.DMA,
        ],
    )
    def sc_gather(f_hbm, src_hbm, out_hbm, idx_v, rows_v, sem):
        wid = lax.axis_index("c") * NSUB + lax.axis_index("s")
        base = wid * RPW
        for ch in range(RPW // GCH):
            start = base + ch * GCH
            pltpu.sync_copy(src_hbm.at[pl.ds(start, GCH)], idx_v)
            pltpu.async_copy(f_hbm.at[idx_v], rows_v, sem).wait()
            pltpu.sync_copy(rows_v, out_hbm.at[pl.ds(start, GCH)])

    @functools.partial(
        pl.kernel,
        out_type=jax.ShapeDtypeStruct((N_PAD, C), jnp.float32),
        mesh=mesh,
        scratch_types=[
            pltpu.VMEM((ZCH, C), jnp.float32),
            pltpu.VMEM((UPS,), jnp.int32),
            pltpu.VMEM((UPS,), jnp.int32),
            pltpu.VMEM((UPS, C), jnp.float32),
            pltpu.VMEM((UPS, C), jnp.float32),
            pltpu.SemaphoreType.DMA,
        ],
    )
    def sc_scatter(corr_hbm, src2_hbm, udst_hbm, out_hbm,
                   zbuf, idx_v, udst_v, acc, tmp, sem):
        cid = lax.axis_index("c")
        sid = lax.axis_index("s")
        wid = cid * NSUB + sid

        zero16 = jnp.zeros((16,), jnp.float32)

        def _zrow(i, _):
            for v in range(C // 16):
                zbuf[i, pl.ds(v * 16, 16)] = zero16
            return 0

        lax.fori_loop(0, ZCH, _zrow, 0)
        zbase = wid * ZPW
        for j in range(ZPW // ZCH):
            pltpu.sync_copy(zbuf, out_hbm.at[pl.ds(zbase + j * ZCH, ZCH)])

        plsc.subcore_barrier()

        ubase = sid * UPS
        pltpu.sync_copy(udst_hbm.at[cid, pl.ds(ubase, UPS)], udst_v)
        pltpu.sync_copy(src2_hbm.at[cid, 0, pl.ds(ubase, UPS)], idx_v)
        pltpu.async_copy(corr_hbm.at[idx_v], acc, sem).wait()
        for r in range(1, R_CAP):
            pltpu.sync_copy(src2_hbm.at[cid, r, pl.ds(ubase, UPS)], idx_v)
            pltpu.async_copy(corr_hbm.at[idx_v], tmp, sem).wait()

            def _arow(i, _):
                for v in range(C // 16):
                    sl = pl.ds(v * 16, 16)
                    acc[i, sl] = acc[i, sl] + tmp[i, sl]
                return 0

            lax.fori_loop(0, UPS, _arow, 0)

        pltpu.async_copy(acc, out_hbm.at[udst_v], sem).wait()

    return sc_search, sc_gather, sc_scatter


def _neighbor_tables(coords):
    ci = coords.astype(jnp.int32)
    keys = ci[:, 0] * (G * G) + ci[:, 1] * G + ci[:, 2]
    order = jnp.argsort(keys)
    skeys = keys[order]
    offs = jnp.array([[dx, dy, dz]
                      for dx in (-1, 0, 1)
                      for dy in (-1, 0, 1)
                      for dz in (-1, 0, 1)], jnp.int32)
    nc = ci[None, :, :] + offs[:, None, :]
    inb = jnp.all((nc >= 0) & (nc < G), axis=2)
    nkey = nc[..., 0] * (G * G) + nc[..., 1] * G + nc[..., 2]
    pos = jnp.clip(jnp.searchsorted(skeys, nkey.reshape(-1)).reshape(27, N),
                   0, N - 1)
    found = inb & (skeys[pos] == nkey)
    src = order[pos]
    return found, src


def _group_by_destination(dst_flat):
    P = ROWS
    sort_idx = jnp.argsort(dst_flat).astype(jnp.int32)
    sdst = dst_flat[sort_idx]
    valid_h = sdst < BIG
    flag = jnp.concatenate([jnp.ones((1,), bool),
                            sdst[1:] != sdst[:-1]]) & valid_h
    ar = jnp.arange(P, dtype=jnp.int32)
    runstart = lax.cummax(jnp.where(flag, ar, 0))
    rpos = ar - runstart
    maxrun = jnp.max(jnp.where(valid_h, rpos, 0)) + 1
    overflow = maxrun > R_CAP

    u0 = jnp.sum((flag & (sdst < HALF)).astype(jnp.int32))
    u_total = jnp.sum(flag.astype(jnp.int32))
    overflow |= (u0 > U_PC) | ((u_total - u0) > U_PC)

    ustart = jnp.argsort(jnp.logical_not(flag), stable=True).astype(jnp.int32)

    gaps = (sdst[1:] > sdst[:-1] + 1) & valid_h[:-1] & (sdst[:-1] + 1 < HALF)
    jj = jnp.argmax(gaps)
    hole0 = jnp.where(sdst[0] != 0, 0, sdst[jj] + 1).astype(jnp.int32)

    sc_ix = jnp.arange(NSC, dtype=jnp.int32)[:, None, None]
    u_ix = jnp.arange(U_PC, dtype=jnp.int32)[None, None, :]
    r_ix = jnp.arange(R_CAP, dtype=jnp.int32)[None, :, None]
    guid = sc_ix * u0 + u_ix
    uvalid = jnp.where(sc_ix == 0, u_ix < u0, guid < u_total)
    st = ustart[jnp.minimum(guid, P - 1)]
    q = st + r_ix
    qc = jnp.minimum(q, P - 1)
    entry_ok = uvalid & (q < P) & (sdst[qc] == sdst[st])
    src2_pk = jnp.where(entry_ok, sort_idx[qc], ZR)
    hole = jnp.stack([hole0, jnp.int32(N)])[:, None]
    udst_pk = jnp.where(uvalid[:, 0, :], sdst[st[:, 0, :]], hole)

    return src2_pk, udst_pk, overflow


def kernel(features, coords, W_conv, b_conv, W_lin, b_lin, gamma, beta):
    ci = coords.astype(jnp.int32)
    keys = ci[:, 0] * (G * G) + ci[:, 1] * G + ci[:, 2]
    order = jnp.argsort(keys).astype(jnp.int32)
    skeys = keys[order]
    kpad = jnp.concatenate(
        [keys, jnp.full((N_PAD - N,), PADKEY, jnp.int32)])
    skp = jnp.concatenate(
        [skeys, jnp.full((SKP - N,), BIG - 1, jnp.int32)])
    ordp = jnp.concatenate(
        [order, jnp.zeros((SKP - N,), jnp.int32)])

    sc_search, sc_gather, sc_scatter = _sc_kernels()
    srcw, dstw, cntw = sc_search(kpad, skp, ordp)
    src_pad = srcw.reshape(NW, NOFF, CAPW).transpose(1, 0, 2).reshape(ROWS)
    dst_flat = dstw.reshape(NW, NOFF, CAPW).transpose(1, 0, 2).reshape(ROWS)
    cnts = cntw.reshape(NW, 32)[:, :NOFF]
    overflow = jnp.any(cnts >= CAPW)
    valid3 = (jnp.arange(CAPW, dtype=jnp.int32)[None, None, :]
              < cnts.T[:, :, None]).astype(jnp.float32).reshape(NOFF, CAP, 1)

    src2_pk, udst_pk, ovf2 = _group_by_destination(dst_flat)
    overflow |= ovf2

    Wf = _wfold(W_conv, W_lin)
    bf = (b_conv @ W_lin + b_lin).reshape(1, C)
    g2 = gamma.reshape(1, C)
    b2 = beta.reshape(1, C)

    def _fast(_):
        gc = sc_gather(features, src_pad)
        corr = _corr_mm(gc.reshape(NOFF, CAP, C), valid3, Wf)
        cbuf = sc_scatter(corr.reshape(ROWS, C), src2_pk, udst_pk)
        return _main(features, cbuf, Wf, bf, g2, b2)

    def _dense(_):
        found, src = _neighbor_tables(coords)
        gath = features[src] * found[..., None].astype(features.dtype)
        h = jnp.einsum("knc,kco->no", gath, W_conv) + b_conv
        h = h @ W_lin + b_lin
        mu = jnp.mean(h, axis=-1, keepdims=True)
        var = jnp.var(h, axis=-1, keepdims=True)
        return (h - mu) / jnp.sqrt(var + 1e-5) * gamma + beta

    return lax.cond(overflow, _dense, _fast, 0)


# speedup vs baseline: 91.4193x; 1.1513x over previous
import functools

import jax
import jax.numpy as jnp
from jax import lax
from jax.experimental import pallas as pl
from jax.experimental.pallas import tpu as pltpu
from jax.experimental.pallas import tpu_sc as plsc

N = 50000
C = 128
G = 256
NOFF = 26
NSC = 2
NSUB = 16
NW = NSC * NSUB
CAPW = 32
CAP = NW * CAPW
ROWS = NOFF * CAP
R_CAP = 4
U_PC = 2048
ZR = CAPW - 1
BIG = 2 ** 30
PADKEY = -(2 ** 20)

RPW = ROWS // NW
GCH = 104
UPS = U_PC // NSUB
N_PAD = 50176
KPW = N_PAD // NW
ZPW = N_PAD // NW
ZCH = 224
HALF = N_PAD // 2
BLK = 1024
SRB = NOFF * CAPW
SRB_PAD = 896
SKP = 50048

OFFS26 = [(dx, dy, dz)
          for dx in (-1, 0, 1) for dy in (-1, 0, 1) for dz in (-1, 0, 1)
          if not (dx == 0 and dy == 0 and dz == 0)]


def _wfold_body(wc_ref, wl_ref, out_ref):
    out_ref[0] = jnp.dot(wc_ref[0], wl_ref[...],
                         preferred_element_type=jnp.float32)


_wfold = pl.pallas_call(
    _wfold_body,
    grid=(27,),
    in_specs=[
        pl.BlockSpec((1, C, C), lambda k: (k, 0, 0)),
        pl.BlockSpec((C, C), lambda k: (0, 0)),
    ],
    out_specs=pl.BlockSpec((1, C, C), lambda k: (k, 0, 0)),
    out_shape=jax.ShapeDtypeStruct((27, C, C), jnp.float32),
)


def _corr_body(gc_ref, valid_ref, wf_ref, out_ref):
    g = gc_ref[0] * valid_ref[0]
    out_ref[0] = jnp.dot(g, wf_ref[0], preferred_element_type=jnp.float32)


_corr_mm = pl.pallas_call(
    _corr_body,
    grid=(NOFF,),
    in_specs=[
        pl.BlockSpec((1, CAP, C), lambda k: (k, 0, 0)),
        pl.BlockSpec((1, CAP, 1), lambda k: (k, 0, 0)),
        pl.BlockSpec((1, C, C), lambda k: (k + (k >= 13).astype(jnp.int32), 0, 0)),
    ],
    out_specs=pl.BlockSpec((1, CAP, C), lambda k: (k, 0, 0)),
    out_shape=jax.ShapeDtypeStruct((NOFF, CAP, C), jnp.float32),
)


def _main_body(f_ref, c_ref, wf_ref, bf_ref, g_ref, b_ref, out_ref):
    h = jnp.dot(f_ref[...], wf_ref[0], preferred_element_type=jnp.float32)
    h = h + c_ref[...] + bf_ref[...]
    mu = jnp.mean(h, axis=1, keepdims=True)
    hc = h - mu
    var = jnp.mean(hc * hc, axis=1, keepdims=True)
    out_ref[...] = hc * lax.rsqrt(var + 1e-5) * g_ref[...] + b_ref[...]


_main = pl.pallas_call(
    _main_body,
    grid=(pl.cdiv(N, BLK),),
    in_specs=[
        pl.BlockSpec((BLK, C), lambda i: (i, 0)),
        pl.BlockSpec((BLK, C), lambda i: (i, 0)),
        pl.BlockSpec((1, C, C), lambda i: (13, 0, 0)),
        pl.BlockSpec((1, C), lambda i: (0, 0)),
        pl.BlockSpec((1, C), lambda i: (0, 0)),
        pl.BlockSpec((1, C), lambda i: (0, 0)),
    ],
    out_specs=pl.BlockSpec((BLK, C), lambda i: (i, 0)),
    out_shape=jax.ShapeDtypeStruct((N, C), jnp.float32),
)


@functools.lru_cache(maxsize=None)
def _sc_kernels():
    mesh = plsc.VectorSubcoreMesh(core_axis_name="c", subcore_axis_name="s",
                                  num_cores=NSC, num_subcores=NSUB)

    @functools.partial(
        pl.kernel,
        out_type=(jax.ShapeDtypeStruct((NW * SRB,), jnp.int32),
                  jax.ShapeDtypeStruct((NW * SRB,), jnp.int32),
                  jax.ShapeDtypeStruct((NW * 32,), jnp.int32)),
        mesh=mesh,
        compiler_params=pltpu.CompilerParams(needs_layout_passes=False),
        scratch_types=[
            pltpu.VMEM((KPW,), jnp.int32),
            pltpu.VMEM((SKP,), jnp.int32),
            pltpu.VMEM((SKP,), jnp.int32),
            pltpu.VMEM((SRB_PAD,), jnp.int32),
            pltpu.VMEM((SRB_PAD,), jnp.int32),
            pltpu.VMEM((32,), jnp.int32),
        ],
    )
    def sc_search(keys_hbm, skeys_hbm, order_hbm,
                  src_hbm, dst_hbm, cnt_hbm,
                  mykeys, skeys_v, order_v, srcb, dstb, cntb):
        wid = lax.axis_index("c") * NSUB + lax.axis_index("s")
        pltpu.sync_copy(keys_hbm.at[pl.ds(wid * KPW, KPW)], mykeys)
        pltpu.sync_copy(skeys_hbm, skeys_v)
        pltpu.sync_copy(order_hbm, order_v)

        zero16 = jnp.zeros((16,), jnp.int32)
        big16 = jnp.full((16,), BIG, jnp.int32)
        for i in range(SRB // 16):
            srcb[pl.ds(i * 16, 16)] = zero16
            dstb[pl.ds(i * 16, 16)] = big16
        iota16 = lax.iota(jnp.int32, 16)

        def row_step(t, cnts):
            key = mykeys[pl.ds(t * 16, 16)]
            dstv = wid * KPW + t * 16 + iota16
            x = key >> 16
            y = (key >> 8) & 255
            z = key & 255
            new_cnts = []
            for j, (dx, dy, dz) in enumerate(OFFS26):
                xx = x + dx
                yy = y + dy
                zz = z + dz
                m = ((xx >= 0) & (xx < G) & (yy >= 0) & (yy < G)
                     & (zz >= 0) & (zz < G))
                nkey = key + (dx * G * G + dy * G + dz)

                def bstep(s, lohi):
                    lo, hi = lohi
                    mid = (lo + hi) >> 1
                    sk = plsc.load_gather(skeys_v, [mid])
                    less = sk < nkey
                    return (jnp.where(less, mid + 1, lo),
                            jnp.where(less, hi, mid))

                lo, _hi = lax.fori_loop(
                    0, 16, bstep,
                    (jnp.zeros((16,), jnp.int32),
                     jnp.full((16,), N, jnp.int32)))
                posc = jnp.minimum(lo, N - 1)
                skp = plsc.load_gather(skeys_v, [posc])
                fnd = m & (lo < N) & (skp == nkey)
                srcv = plsc.load_gather(order_v, [posc])
                fi = fnd.astype(jnp.int32)
                csum = plsc.cumsum(fi)
                posb = j * CAPW + cnts[j] + csum - fi
                okw = fnd & (csum - fi + cnts[j] < CAPW)
                plsc.store_scatter(srcb, [posb], srcv, mask=okw)
                plsc.store_scatter(dstb, [posb], dstv, mask=okw)
                new_cnts.append(cnts[j] + jnp.sum(fi))
            return tuple(new_cnts)

        cnts_fin = lax.fori_loop(0, KPW // 16, row_step,
                                 tuple(jnp.int32(0) for _ in range(NOFF)))

        c0 = zero16
        c1 = zero16
        for k in range(16):
            c0 = jnp.where(iota16 == k, cnts_fin[k], c0)
        for k in range(16, NOFF):
            c1 = jnp.where(iota16 == (k - 16), cnts_fin[k], c1)
        cntb[pl.ds(0, 16)] = c0
        cntb[pl.ds(16, 16)] = c1

        pltpu.sync_copy(srcb.at[pl.ds(0, SRB)],
                        src_hbm.at[pl.ds(wid * SRB, SRB)])
        pltpu.sync_copy(dstb.at[pl.ds(0, SRB)],
                        dst_hbm.at[pl.ds(wid * SRB, SRB)])
        pltpu.sync_copy(cntb, cnt_hbm.at[pl.ds(wid * 32, 32)])

    @functools.partial(
        pl.kernel,
        out_type=jax.ShapeDtypeStruct((ROWS, C), jnp.float32),
        mesh=mesh,
        scratch_types=[
            pltpu.VMEM((RPW,), jnp.int32),
            pltpu.VMEM((RPW, C), jnp.float32),
            pltpu.SemaphoreType.DMA,
        ],
    )
    def sc_gather(f_hbm, src_hbm, out_hbm, idx_v, rows_v, sem):
        wid = lax.axis_index("c") * NSUB + lax.axis_index("s")
        base = wid * RPW
        pltpu.sync_copy(src_hbm.at[pl.ds(base, RPW)], idx_v)
        copies = []
        for ch in range(RPW // GCH):
            sl = pl.ds(ch * GCH, GCH)
            copies.append(pltpu.async_copy(
                f_hbm.at[idx_v.at[sl]], rows_v.at[sl], sem))
        for cp in copies:
            cp.wait()
        pltpu.sync_copy(rows_v, out_hbm.at[pl.ds(base, RPW)])

    @functools.partial(
        pl.kernel,
        out_type=jax.ShapeDtypeStruct((N_PAD, C), jnp.float32),
        mesh=mesh,
        scratch_types=[
            pltpu.VMEM((ZCH, C), jnp.float32),
            pltpu.VMEM((R_CAP, UPS), jnp.int32),
            pltpu.VMEM((UPS,), jnp.int32),
            pltpu.VMEM((UPS, C), jnp.float32),
            pltpu.VMEM(((R_CAP - 1), UPS, C), jnp.float32),
            pltpu.SemaphoreType.DMA,
        ],
    )
    def sc_scatter(corr_hbm, src2_hbm, udst_hbm, out_hbm,
                   zbuf, idx_v, udst_v, acc, tmp, sem):
        cid = lax.axis_index("c")
        sid = lax.axis_index("s")
        wid = cid * NSUB + sid

        zero16 = jnp.zeros((16,), jnp.float32)

        def _zrow(i, _):
            for v in range(C // 16):
                zbuf[i, pl.ds(v * 16, 16)] = zero16
            return 0

        lax.fori_loop(0, ZCH, _zrow, 0)
        zbase = wid * ZPW
        zcopies = [pltpu.async_copy(
            zbuf, out_hbm.at[pl.ds(zbase + j * ZCH, ZCH)], sem)
            for j in range(ZPW // ZCH)]
        for cp in zcopies:
            cp.wait()

        plsc.subcore_barrier()

        ubase = sid * UPS
        pltpu.sync_copy(udst_hbm.at[cid, pl.ds(ubase, UPS)], udst_v)
        pltpu.sync_copy(src2_hbm.at[cid, :, pl.ds(ubase, UPS)], idx_v)
        rcopies = [pltpu.async_copy(corr_hbm.at[idx_v.at[0]], acc, sem)]
        for r in range(1, R_CAP):
            rcopies.append(pltpu.async_copy(
                corr_hbm.at[idx_v.at[r]], tmp.at[r - 1], sem))
        for cp in rcopies:
            cp.wait()
        for r in range(1, R_CAP):

            def _arow(i, _):
                for v in range(C // 16):
                    sl = pl.ds(v * 16, 16)
                    acc[i, sl] = acc[i, sl] + tmp[r - 1, i, sl]
                return 0

            lax.fori_loop(0, UPS, _arow, 0)

        pltpu.async_copy(acc, out_hbm.at[udst_v], sem).wait()

    return sc_search, sc_gather, sc_scatter


def _neighbor_tables(coords):
    ci = coords.astype(jnp.int32)
    keys = ci[:, 0] * (G * G) + ci[:, 1] * G + ci[:, 2]
    order = jnp.argsort(keys)
    skeys = keys[order]
    offs = jnp.array([[dx, dy, dz]
                      for dx in (-1, 0, 1)
                      for dy in (-1, 0, 1)
                      for dz in (-1, 0, 1)], jnp.int32)
    nc = ci[None, :, :] + offs[:, None, :]
    inb = jnp.all((nc >= 0) & (nc < G), axis=2)
    nkey = nc[..., 0] * (G * G) + nc[..., 1] * G + nc[..., 2]
    pos = jnp.clip(jnp.searchsorted(skeys, nkey.reshape(-1)).reshape(27, N),
                   0, N - 1)
    found = inb & (skeys[pos] == nkey)
    src = order[pos]
    return found, src


def _group_by_destination(dst_flat):
    P = ROWS
    sort_idx = jnp.argsort(dst_flat).astype(jnp.int32)
    sdst = dst_flat[sort_idx]
    valid_h = sdst < BIG
    flag = jnp.concatenate([jnp.ones((1,), bool),
                            sdst[1:] != sdst[:-1]]) & valid_h
    ar = jnp.arange(P, dtype=jnp.int32)
    runstart = lax.cummax(jnp.where(flag, ar, 0))
    rpos = ar - runstart
    maxrun = jnp.max(jnp.where(valid_h, rpos, 0)) + 1
    overflow = maxrun > R_CAP

    u0 = jnp.sum((flag & (sdst < HALF)).astype(jnp.int32))
    u_total = jnp.sum(flag.astype(jnp.int32))
    overflow |= (u0 > U_PC) | ((u_total - u0) > U_PC)

    ustart = jnp.argsort(jnp.logical_not(flag), stable=True).astype(jnp.int32)

    gaps = (sdst[1:] > sdst[:-1] + 1) & valid_h[:-1] & (sdst[:-1] + 1 < HALF)
    jj = jnp.argmax(gaps)
    hole0 = jnp.where(sdst[0] != 0, 0, sdst[jj] + 1).astype(jnp.int32)

    sc_ix = jnp.arange(NSC, dtype=jnp.int32)[:, None, None]
    u_ix = jnp.arange(U_PC, dtype=jnp.int32)[None, None, :]
    r_ix = jnp.arange(R_CAP, dtype=jnp.int32)[None, :, None]
    guid = sc_ix * u0 + u_ix
    uvalid = jnp.where(sc_ix == 0, u_ix < u0, guid < u_total)
    st = ustart[jnp.minimum(guid, P - 1)]
    q = st + r_ix
    qc = jnp.minimum(q, P - 1)
    entry_ok = uvalid & (q < P) & (sdst[qc] == sdst[st])
    src2_pk = jnp.where(entry_ok, sort_idx[qc], ZR)
    hole = jnp.stack([hole0, jnp.int32(N)])[:, None]
    udst_pk = jnp.where(uvalid[:, 0, :], sdst[st[:, 0, :]], hole)

    return src2_pk, udst_pk, overflow


def kernel(features, coords, W_conv, b_conv, W_lin, b_lin, gamma, beta):
    ci = coords.astype(jnp.int32)
    keys = ci[:, 0] * (G * G) + ci[:, 1] * G + ci[:, 2]
    order = jnp.argsort(keys).astype(jnp.int32)
    skeys = keys[order]
    kpad = jnp.concatenate(
        [keys, jnp.full((N_PAD - N,), PADKEY, jnp.int32)])
    skp = jnp.concatenate(
        [skeys, jnp.full((SKP - N,), BIG - 1, jnp.int32)])
    ordp = jnp.concatenate(
        [order, jnp.zeros((SKP - N,), jnp.int32)])

    sc_search, sc_gather, sc_scatter = _sc_kernels()
    srcw, dstw, cntw = sc_search(kpad, skp, ordp)
    src_pad = srcw.reshape(NW, NOFF, CAPW).transpose(1, 0, 2).reshape(ROWS)
    dst_flat = dstw.reshape(NW, NOFF, CAPW).transpose(1, 0, 2).reshape(ROWS)
    cnts = cntw.reshape(NW, 32)[:, :NOFF]
    overflow = jnp.any(cnts >= CAPW)
    valid3 = (jnp.arange(CAPW, dtype=jnp.int32)[None, None, :]
              < cnts.T[:, :, None]).astype(jnp.float32).reshape(NOFF, CAP, 1)

    src2_pk, udst_pk, ovf2 = _group_by_destination(dst_flat)
    overflow |= ovf2

    Wf = _wfold(W_conv, W_lin)
    bf = (b_conv @ W_lin + b_lin).reshape(1, C)
    g2 = gamma.reshape(1, C)
    b2 = beta.reshape(1, C)

    def _fast(_):
        gc = sc_gather(features, src_pad)
        corr = _corr_mm(gc.reshape(NOFF, CAP, C), valid3, Wf)
        cbuf = sc_scatter(corr.reshape(ROWS, C), src2_pk, udst_pk)
        return _main(features, cbuf, Wf, bf, g2, b2)

    def _dense(_):
        found, src = _neighbor_tables(coords)
        gath = features[src] * found[..., None].astype(features.dtype)
        h = jnp.einsum("knc,kco->no", gath, W_conv) + b_conv
        h = h @ W_lin + b_lin
        mu = jnp.mean(h, axis=-1, keepdims=True)
        var = jnp.var(h, axis=-1, keepdims=True)
        return (h - mu) / jnp.sqrt(var + 1e-5) * gamma + beta

    return lax.cond(overflow, _dense, _fast, 0)


# speedup vs baseline: 147.2898x; 1.6111x over previous
import functools

import jax
import jax.numpy as jnp
from jax import lax
from jax.experimental import pallas as pl
from jax.experimental.pallas import tpu as pltpu
from jax.experimental.pallas import tpu_sc as plsc

N = 50000
C = 128
G = 256
NOFF = 26
NSC = 2
NSUB = 16
NW = NSC * NSUB
CAPW = 16
CAP = NW * CAPW
ROWS = NOFF * CAP
R_CAP = 4
U_PC = 2048
ZR = CAPW - 1
BIG = 2 ** 30
PADKEY = -(2 ** 20)

RPW = ROWS // NW
GCH = 104
UPS = U_PC // NSUB
N_PAD = 50176
KPW = N_PAD // NW
ZPW = N_PAD // NW
ZCH = 224
HALF = N_PAD // 2
BLK = 1024
SRB = NOFF * CAPW
SRB_PAD = 512
SKP = 50048

OFFS26 = [(dx, dy, dz)
          for dx in (-1, 0, 1) for dy in (-1, 0, 1) for dz in (-1, 0, 1)
          if not (dx == 0 and dy == 0 and dz == 0)]


def _wfold_body(wc_ref, wl_ref, out_ref):
    out_ref[0] = jnp.dot(wc_ref[0], wl_ref[...],
                         preferred_element_type=jnp.float32)


_wfold = pl.pallas_call(
    _wfold_body,
    grid=(27,),
    in_specs=[
        pl.BlockSpec((1, C, C), lambda k: (k, 0, 0)),
        pl.BlockSpec((C, C), lambda k: (0, 0)),
    ],
    out_specs=pl.BlockSpec((1, C, C), lambda k: (k, 0, 0)),
    out_shape=jax.ShapeDtypeStruct((27, C, C), jnp.float32),
)


def _corr_body(gc_ref, valid_ref, wf_ref, out_ref):
    g = gc_ref[0] * valid_ref[0]
    out_ref[0] = jnp.dot(g, wf_ref[0], preferred_element_type=jnp.float32)


_corr_mm = pl.pallas_call(
    _corr_body,
    grid=(NOFF,),
    in_specs=[
        pl.BlockSpec((1, CAP, C), lambda k: (k, 0, 0)),
        pl.BlockSpec((1, CAP, 1), lambda k: (k, 0, 0)),
        pl.BlockSpec((1, C, C), lambda k: (k + (k >= 13).astype(jnp.int32), 0, 0)),
    ],
    out_specs=pl.BlockSpec((1, CAP, C), lambda k: (k, 0, 0)),
    out_shape=jax.ShapeDtypeStruct((NOFF, CAP, C), jnp.float32),
)


def _main_body(f_ref, c_ref, wf_ref, bf_ref, g_ref, b_ref, out_ref):
    h = jnp.dot(f_ref[...], wf_ref[0], preferred_element_type=jnp.float32)
    h = h + c_ref[...] + bf_ref[...]
    mu = jnp.mean(h, axis=1, keepdims=True)
    hc = h - mu
    var = jnp.mean(hc * hc, axis=1, keepdims=True)
    out_ref[...] = hc * lax.rsqrt(var + 1e-5) * g_ref[...] + b_ref[...]


_main = pl.pallas_call(
    _main_body,
    grid=(pl.cdiv(N, BLK),),
    in_specs=[
        pl.BlockSpec((BLK, C), lambda i: (i, 0)),
        pl.BlockSpec((BLK, C), lambda i: (i, 0)),
        pl.BlockSpec((1, C, C), lambda i: (13, 0, 0)),
        pl.BlockSpec((1, C), lambda i: (0, 0)),
        pl.BlockSpec((1, C), lambda i: (0, 0)),
        pl.BlockSpec((1, C), lambda i: (0, 0)),
    ],
    out_specs=pl.BlockSpec((BLK, C), lambda i: (i, 0)),
    out_shape=jax.ShapeDtypeStruct((N, C), jnp.float32),
)


@functools.lru_cache(maxsize=None)
def _sc_kernels():
    mesh = plsc.VectorSubcoreMesh(core_axis_name="c", subcore_axis_name="s",
                                  num_cores=NSC, num_subcores=NSUB)

    @functools.partial(
        pl.kernel,
        out_type=(jax.ShapeDtypeStruct((NW * SRB,), jnp.int32),
                  jax.ShapeDtypeStruct((NW * SRB,), jnp.int32),
                  jax.ShapeDtypeStruct((NW * 32,), jnp.int32)),
        mesh=mesh,
        compiler_params=pltpu.CompilerParams(needs_layout_passes=False),
        scratch_types=[
            pltpu.VMEM((KPW,), jnp.int32),
            pltpu.VMEM((SKP,), jnp.int32),
            pltpu.VMEM((SKP,), jnp.int32),
            pltpu.VMEM((SRB_PAD,), jnp.int32),
            pltpu.VMEM((SRB_PAD,), jnp.int32),
            pltpu.VMEM((32,), jnp.int32),
        ],
    )
    def sc_search(keys_hbm, skeys_hbm, order_hbm,
                  src_hbm, dst_hbm, cnt_hbm,
                  mykeys, skeys_v, order_v, srcb, dstb, cntb):
        wid = lax.axis_index("c") * NSUB + lax.axis_index("s")
        pltpu.sync_copy(keys_hbm.at[pl.ds(wid * KPW, KPW)], mykeys)
        pltpu.sync_copy(skeys_hbm, skeys_v)
        pltpu.sync_copy(order_hbm, order_v)

        zero16 = jnp.zeros((16,), jnp.int32)
        big16 = jnp.full((16,), BIG, jnp.int32)
        for i in range(SRB // 16):
            srcb[pl.ds(i * 16, 16)] = zero16
            dstb[pl.ds(i * 16, 16)] = big16
        iota16 = lax.iota(jnp.int32, 16)

        def row_step(t, cnts):
            key = mykeys[pl.ds(t * 16, 16)]
            dstv = wid * KPW + t * 16 + iota16
            x = key >> 16
            y = (key >> 8) & 255
            z = key & 255
            new_cnts = []
            for j, (dx, dy, dz) in enumerate(OFFS26):
                xx = x + dx
                yy = y + dy
                zz = z + dz
                m = ((xx >= 0) & (xx < G) & (yy >= 0) & (yy < G)
                     & (zz >= 0) & (zz < G))
                nkey = key + (dx * G * G + dy * G + dz)

                def bstep(s, lohi):
                    lo, hi = lohi
                    mid = (lo + hi) >> 1
                    sk = plsc.load_gather(skeys_v, [mid])
                    less = sk < nkey
                    return (jnp.where(less, mid + 1, lo),
                            jnp.where(less, hi, mid))

                lo, _hi = lax.fori_loop(
                    0, 16, bstep,
                    (jnp.zeros((16,), jnp.int32),
                     jnp.full((16,), N, jnp.int32)))
                posc = jnp.minimum(lo, N - 1)
                skp = plsc.load_gather(skeys_v, [posc])
                fnd = m & (lo < N) & (skp == nkey)
                srcv = plsc.load_gather(order_v, [posc])
                fi = fnd.astype(jnp.int32)
                csum = plsc.cumsum(fi)
                posb = j * CAPW + cnts[j] + csum - fi
                okw = fnd & (csum - fi + cnts[j] < CAPW)
                plsc.store_scatter(srcb, [posb], srcv, mask=okw)
                plsc.store_scatter(dstb, [posb], dstv, mask=okw)
                new_cnts.append(cnts[j] + jnp.sum(fi))
            return tuple(new_cnts)

        cnts_fin = lax.fori_loop(0, KPW // 16, row_step,
                                 tuple(jnp.int32(0) for _ in range(NOFF)))

        c0 = zero16
        c1 = zero16
        for k in range(16):
            c0 = jnp.where(iota16 == k, cnts_fin[k], c0)
        for k in range(16, NOFF):
            c1 = jnp.where(iota16 == (k - 16), cnts_fin[k], c1)
        cntb[pl.ds(0, 16)] = c0
        cntb[pl.ds(16, 16)] = c1

        pltpu.sync_copy(srcb.at[pl.ds(0, SRB)],
                        src_hbm.at[pl.ds(wid * SRB, SRB)])
        pltpu.sync_copy(dstb.at[pl.ds(0, SRB)],
                        dst_hbm.at[pl.ds(wid * SRB, SRB)])
        pltpu.sync_copy(cntb, cnt_hbm.at[pl.ds(wid * 32, 32)])

    @functools.partial(
        pl.kernel,
        out_type=jax.ShapeDtypeStruct((ROWS, C), jnp.float32),
        mesh=mesh,
        compiler_params=pltpu.CompilerParams(needs_layout_passes=False),
        scratch_types=[
            pltpu.VMEM((RPW,), jnp.int32),
            pltpu.VMEM((RPW, C), jnp.float32),
            pltpu.SemaphoreType.DMA,
        ],
    )
    def sc_gather(f_hbm, src_hbm, out_hbm, idx_v, rows_v, sem):
        wid = lax.axis_index("c") * NSUB + lax.axis_index("s")
        base = wid * RPW
        pltpu.sync_copy(src_hbm.at[pl.ds(base, RPW)], idx_v)
        copies = []
        for ch in range(RPW // GCH):
            sl = pl.ds(ch * GCH, GCH)
            copies.append(pltpu.async_copy(
                f_hbm.at[idx_v.at[sl]], rows_v.at[sl], sem))
        for cp in copies:
            cp.wait()
        pltpu.sync_copy(rows_v, out_hbm.at[pl.ds(base, RPW)])

    @functools.partial(
        pl.kernel,
        out_type=jax.ShapeDtypeStruct((N_PAD, C), jnp.float32),
        mesh=mesh,
        compiler_params=pltpu.CompilerParams(needs_layout_passes=False),
        scratch_types=[
            pltpu.VMEM((ZCH, C), jnp.float32),
            pltpu.VMEM((R_CAP, UPS), jnp.int32),
            pltpu.VMEM((UPS,), jnp.int32),
            pltpu.VMEM((UPS, C), jnp.float32),
            pltpu.VMEM((1, UPS, C), jnp.float32),
            pltpu.SemaphoreType.DMA,
        ],
    )
    def sc_scatter(corr_hbm, src2_hbm, udst_hbm, out_hbm,
                   zbuf, idx_v, udst_v, acc, tmp, sem):
        cid = lax.axis_index("c")
        sid = lax.axis_index("s")
        wid = cid * NSUB + sid

        zero16 = jnp.zeros((16,), jnp.float32)

        def _zrow(i, _):
            for v in range(C // 16):
                zbuf[i, pl.ds(v * 16, 16)] = zero16
            return 0

        lax.fori_loop(0, ZCH, _zrow, 0)
        zbase = wid * ZPW
        zcopies = [pltpu.async_copy(
            zbuf, out_hbm.at[pl.ds(zbase + j * ZCH, ZCH)], sem)
            for j in range(ZPW // ZCH)]
        for cp in zcopies:
            cp.wait()

        plsc.subcore_barrier()

        ubase = sid * UPS
        pltpu.sync_copy(udst_hbm.at[cid, pl.ds(ubase, UPS)], udst_v)
        pltpu.sync_copy(src2_hbm.at[cid, :, pl.ds(ubase, UPS)], idx_v)

        def _add_tmp():
            def _arow(i, _):
                for v in range(C // 16):
                    sl = pl.ds(v * 16, 16)
                    acc[i, sl] = acc[i, sl] + tmp[0, i, sl]
                return 0

            lax.fori_loop(0, UPS, _arow, 0)

        c0 = pltpu.async_copy(corr_hbm.at[idx_v.at[0]], acc, sem)
        c1 = pltpu.async_copy(corr_hbm.at[idx_v.at[1]], tmp.at[0], sem)
        c0.wait()
        c1.wait()
        _add_tmp()
        for r in range(2, R_CAP):
            nz = jnp.zeros((16,), jnp.int32)
            for v in range(UPS // 16):
                chunk = idx_v[r, pl.ds(v * 16, 16)]
                nz = jnp.maximum(nz, jnp.where(chunk != ZR, 1, 0))

            @pl.when(jnp.max(nz) > 0)
            def _round():
                pltpu.async_copy(
                    corr_hbm.at[idx_v.at[r]], tmp.at[0], sem).wait()
                _add_tmp()

        pltpu.async_copy(acc, out_hbm.at[udst_v], sem).wait()

    return sc_search, sc_gather, sc_scatter


def _neighbor_tables(coords):
    ci = coords.astype(jnp.int32)
    keys = ci[:, 0] * (G * G) + ci[:, 1] * G + ci[:, 2]
    order = jnp.argsort(keys)
    skeys = keys[order]
    offs = jnp.array([[dx, dy, dz]
                      for dx in (-1, 0, 1)
                      for dy in (-1, 0, 1)
                      for dz in (-1, 0, 1)], jnp.int32)
    nc = ci[None, :, :] + offs[:, None, :]
    inb = jnp.all((nc >= 0) & (nc < G), axis=2)
    nkey = nc[..., 0] * (G * G) + nc[..., 1] * G + nc[..., 2]
    pos = jnp.clip(jnp.searchsorted(skeys, nkey.reshape(-1)).reshape(27, N),
                   0, N - 1)
    found = inb & (skeys[pos] == nkey)
    src = order[pos]
    return found, src


def _group_by_destination(dst_flat):
    P = ROWS
    sort_idx = jnp.argsort(dst_flat).astype(jnp.int32)
    sdst = dst_flat[sort_idx]
    valid_h = sdst < BIG
    flag = jnp.concatenate([jnp.ones((1,), bool),
                            sdst[1:] != sdst[:-1]]) & valid_h
    ar = jnp.arange(P, dtype=jnp.int32)
    runstart = lax.cummax(jnp.where(flag, ar, 0))
    rpos = ar - runstart
    maxrun = jnp.max(jnp.where(valid_h, rpos, 0)) + 1
    overflow = maxrun > R_CAP

    u0 = jnp.sum((flag & (sdst < HALF)).astype(jnp.int32))
    u_total = jnp.sum(flag.astype(jnp.int32))
    overflow |= (u0 > U_PC) | ((u_total - u0) > U_PC)

    ustart = jnp.argsort(jnp.logical_not(flag), stable=True).astype(jnp.int32)

    gaps = (sdst[1:] > sdst[:-1] + 1) & valid_h[:-1] & (sdst[:-1] + 1 < HALF)
    jj = jnp.argmax(gaps)
    hole0 = jnp.where(sdst[0] != 0, 0, sdst[jj] + 1).astype(jnp.int32)

    sc_ix = jnp.arange(NSC, dtype=jnp.int32)[:, None, None]
    u_ix = jnp.arange(U_PC, dtype=jnp.int32)[None, None, :]
    r_ix = jnp.arange(R_CAP, dtype=jnp.int32)[None, :, None]
    guid = sc_ix * u0 + u_ix
    uvalid = jnp.where(sc_ix == 0, u_ix < u0, guid < u_total)
    st = ustart[jnp.minimum(guid, P - 1)]
    q = st + r_ix
    qc = jnp.minimum(q, P - 1)
    entry_ok = uvalid & (q < P) & (sdst[qc] == sdst[st])
    src2_pk = jnp.where(entry_ok, sort_idx[qc], ZR)
    hole = jnp.stack([hole0, jnp.int32(N)])[:, None]
    udst_pk = jnp.where(uvalid[:, 0, :], sdst[st[:, 0, :]], hole)

    return src2_pk, udst_pk, overflow


def kernel(features, coords, W_conv, b_conv, W_lin, b_lin, gamma, beta):
    ci = coords.astype(jnp.int32)
    keys = ci[:, 0] * (G * G) + ci[:, 1] * G + ci[:, 2]
    order = jnp.argsort(keys).astype(jnp.int32)
    skeys = keys[order]
    kpad = jnp.concatenate(
        [keys, jnp.full((N_PAD - N,), PADKEY, jnp.int32)])
    skp = jnp.concatenate(
        [skeys, jnp.full((SKP - N,), BIG - 1, jnp.int32)])
    ordp = jnp.concatenate(
        [order, jnp.zeros((SKP - N,), jnp.int32)])

    sc_search, sc_gather, sc_scatter = _sc_kernels()
    srcw, dstw, cntw = sc_search(kpad, skp, ordp)
    src_pad = srcw.reshape(NW, NOFF, CAPW).transpose(1, 0, 2).reshape(ROWS)
    dst_flat = dstw.reshape(NW, NOFF, CAPW).transpose(1, 0, 2).reshape(ROWS)
    cnts = cntw.reshape(NW, 32)[:, :NOFF]
    overflow = jnp.any(cnts >= CAPW)
    valid3 = (jnp.arange(CAPW, dtype=jnp.int32)[None, None, :]
              < cnts.T[:, :, None]).astype(jnp.float32).reshape(NOFF, CAP, 1)

    src2_pk, udst_pk, ovf2 = _group_by_destination(dst_flat)
    overflow |= ovf2

    Wf = _wfold(W_conv, W_lin)
    bf = (b_conv @ W_lin + b_lin).reshape(1, C)
    g2 = gamma.reshape(1, C)
    b2 = beta.reshape(1, C)

    def _fast(_):
        gc = sc_gather(features, src_pad)
        corr = _corr_mm(gc.reshape(NOFF, CAP, C), valid3, Wf)
        cbuf = sc_scatter(corr.reshape(ROWS, C), src2_pk, udst_pk)
        return _main(features, cbuf, Wf, bf, g2, b2)

    def _dense(_):
        found, src = _neighbor_tables(coords)
        gath = features[src] * found[..., None].astype(features.dtype)
        h = jnp.einsum("knc,kco->no", gath, W_conv) + b_conv
        h = h @ W_lin + b_lin
        mu = jnp.mean(h, axis=-1, keepdims=True)
        var = jnp.var(h, axis=-1, keepdims=True)
        return (h - mu) / jnp.sqrt(var + 1e-5) * gamma + beta

    return lax.cond(overflow, _dense, _fast, 0)


# speedup vs baseline: 161.5947x; 1.0971x over previous
import functools

import jax
import jax.numpy as jnp
from jax import lax
from jax.experimental import pallas as pl
from jax.experimental.pallas import tpu as pltpu
from jax.experimental.pallas import tpu_sc as plsc

N = 50000
C = 128
G = 256
NOFF = 26
NSC = 2
NSUB = 16
NW = NSC * NSUB
CAPW = 16
CAP = NW * CAPW
ROWS = NOFF * CAP
R_CAP = 4
U_PC = 2048
ZR = CAPW - 1
BIG = 2 ** 30
PADKEY = -(2 ** 20)

RPW = ROWS // NW
GCH = 104
UPS = U_PC // NSUB
N_PAD = 50176
KPW = N_PAD // NW
ZPW = N_PAD // NW
ZCH = 224
HALF = N_PAD // 2
BLK = 1024
SRB = NOFF * CAPW
SRB_PAD = 512
SKP = 50048
NSLAB = 512
SLAB_SH = 15
XSP = 640
BSTEPS = 8

OFFS26 = [(dx, dy, dz)
          for dx in (-1, 0, 1) for dy in (-1, 0, 1) for dz in (-1, 0, 1)
          if not (dx == 0 and dy == 0 and dz == 0)]


def _wfold_body(wc_ref, wl_ref, out_ref):
    out_ref[0] = jnp.dot(wc_ref[0], wl_ref[...],
                         preferred_element_type=jnp.float32)


_wfold = pl.pallas_call(
    _wfold_body,
    grid=(27,),
    in_specs=[
        pl.BlockSpec((1, C, C), lambda k: (k, 0, 0)),
        pl.BlockSpec((C, C), lambda k: (0, 0)),
    ],
    out_specs=pl.BlockSpec((1, C, C), lambda k: (k, 0, 0)),
    out_shape=jax.ShapeDtypeStruct((27, C, C), jnp.float32),
)


def _corr_body(gc_ref, valid_ref, wf_ref, out_ref):
    g = gc_ref[0] * valid_ref[0]
    out_ref[0] = jnp.dot(g, wf_ref[0], preferred_element_type=jnp.float32)


_corr_mm = pl.pallas_call(
    _corr_body,
    grid=(NOFF,),
    in_specs=[
        pl.BlockSpec((1, CAP, C), lambda k: (k, 0, 0)),
        pl.BlockSpec((1, CAP, 1), lambda k: (k, 0, 0)),
        pl.BlockSpec((1, C, C), lambda k: (k + (k >= 13).astype(jnp.int32), 0, 0)),
    ],
    out_specs=pl.BlockSpec((1, CAP, C), lambda k: (k, 0, 0)),
    out_shape=jax.ShapeDtypeStruct((NOFF, CAP, C), jnp.float32),
)


def _main_body(f_ref, c_ref, wf_ref, bf_ref, g_ref, b_ref, out_ref):
    h = jnp.dot(f_ref[...], wf_ref[0], preferred_element_type=jnp.float32)
    h = h + c_ref[...] + bf_ref[...]
    mu = jnp.mean(h, axis=1, keepdims=True)
    hc = h - mu
    var = jnp.mean(hc * hc, axis=1, keepdims=True)
    out_ref[...] = hc * lax.rsqrt(var + 1e-5) * g_ref[...] + b_ref[...]


_main = pl.pallas_call(
    _main_body,
    grid=(pl.cdiv(N, BLK),),
    in_specs=[
        pl.BlockSpec((BLK, C), lambda i: (i, 0)),
        pl.BlockSpec((BLK, C), lambda i: (i, 0)),
        pl.BlockSpec((1, C, C), lambda i: (13, 0, 0)),
        pl.BlockSpec((1, C), lambda i: (0, 0)),
        pl.BlockSpec((1, C), lambda i: (0, 0)),
        pl.BlockSpec((1, C), lambda i: (0, 0)),
    ],
    out_specs=pl.BlockSpec((BLK, C), lambda i: (i, 0)),
    out_shape=jax.ShapeDtypeStruct((N, C), jnp.float32),
)


@functools.lru_cache(maxsize=None)
def _sc_kernels():
    mesh = plsc.VectorSubcoreMesh(core_axis_name="c", subcore_axis_name="s",
                                  num_cores=NSC, num_subcores=NSUB)

    @functools.partial(
        pl.kernel,
        out_type=(jax.ShapeDtypeStruct((NW * SRB,), jnp.int32),
                  jax.ShapeDtypeStruct((NW * SRB,), jnp.int32),
                  jax.ShapeDtypeStruct((NW * 32,), jnp.int32)),
        mesh=mesh,
        compiler_params=pltpu.CompilerParams(needs_layout_passes=False),
        scratch_types=[
            pltpu.VMEM((KPW,), jnp.int32),
            pltpu.VMEM((SKP,), jnp.int32),
            pltpu.VMEM((SKP,), jnp.int32),
            pltpu.VMEM((XSP,), jnp.int32),
            pltpu.VMEM((SRB_PAD,), jnp.int32),
            pltpu.VMEM((SRB_PAD,), jnp.int32),
            pltpu.VMEM((32,), jnp.int32),
        ],
    )
    def sc_search(keys_hbm, skeys_hbm, order_hbm, xslab_hbm,
                  src_hbm, dst_hbm, cnt_hbm,
                  mykeys, skeys_v, order_v, xslab_v, srcb, dstb, cntb):
        wid = lax.axis_index("c") * NSUB + lax.axis_index("s")
        pltpu.sync_copy(keys_hbm.at[pl.ds(wid * KPW, KPW)], mykeys)
        pltpu.sync_copy(skeys_hbm, skeys_v)
        pltpu.sync_copy(order_hbm, order_v)
        pltpu.sync_copy(xslab_hbm, xslab_v)

        zero16 = jnp.zeros((16,), jnp.int32)
        big16 = jnp.full((16,), BIG, jnp.int32)
        for i in range(SRB // 16):
            srcb[pl.ds(i * 16, 16)] = zero16
            dstb[pl.ds(i * 16, 16)] = big16
        iota16 = lax.iota(jnp.int32, 16)

        def row_step(t, cnts):
            key = mykeys[pl.ds(t * 16, 16)]
            dstv = wid * KPW + t * 16 + iota16
            x = key >> 16
            y = (key >> 8) & 255
            z = key & 255
            new_cnts = []
            for j, (dx, dy, dz) in enumerate(OFFS26):
                xx = x + dx
                yy = y + dy
                zz = z + dz
                m = ((xx >= 0) & (xx < G) & (yy >= 0) & (yy < G)
                     & (zz >= 0) & (zz < G))
                nkey = key + (dx * G * G + dy * G + dz)

                def bstep(s, lohi):
                    lo, hi = lohi
                    mid = (lo + hi) >> 1
                    sk = plsc.load_gather(skeys_v, [mid])
                    less = sk < nkey
                    return (jnp.where(less, mid + 1, lo),
                            jnp.where(less, hi, mid))

                bq = jnp.clip(nkey >> SLAB_SH, 0, NSLAB - 1)
                lo0 = plsc.load_gather(xslab_v, [bq])
                hi0 = plsc.load_gather(xslab_v, [bq + 1])
                lo, _hi = lax.fori_loop(0, BSTEPS, bstep, (lo0, hi0))
                posc = jnp.minimum(lo, N - 1)
                skp = plsc.load_gather(skeys_v, [posc])
                fnd = m & (lo < N) & (skp == nkey)
                srcv = plsc.load_gather(order_v, [posc])
                fi = fnd.astype(jnp.int32)
                csum = plsc.cumsum(fi)
                posb = j * CAPW + cnts[j] + csum - fi
                okw = fnd & (csum - fi + cnts[j] < CAPW)
                plsc.store_scatter(srcb, [posb], srcv, mask=okw)
                plsc.store_scatter(dstb, [posb], dstv, mask=okw)
                new_cnts.append(cnts[j] + jnp.sum(fi))
            return tuple(new_cnts)

        cnts_fin = lax.fori_loop(0, KPW // 16, row_step,
                                 tuple(jnp.int32(0) for _ in range(NOFF)))

        c0 = zero16
        c1 = zero16
        for k in range(16):
            c0 = jnp.where(iota16 == k, cnts_fin[k], c0)
        for k in range(16, NOFF):
            c1 = jnp.where(iota16 == (k - 16), cnts_fin[k], c1)
        cntb[pl.ds(0, 16)] = c0
        cntb[pl.ds(16, 16)] = c1

        pltpu.sync_copy(srcb.at[pl.ds(0, SRB)],
                        src_hbm.at[pl.ds(wid * SRB, SRB)])
        pltpu.sync_copy(dstb.at[pl.ds(0, SRB)],
                        dst_hbm.at[pl.ds(wid * SRB, SRB)])
        pltpu.sync_copy(cntb, cnt_hbm.at[pl.ds(wid * 32, 32)])

    @functools.partial(
        pl.kernel,
        out_type=jax.ShapeDtypeStruct((ROWS, C), jnp.float32),
        mesh=mesh,
        compiler_params=pltpu.CompilerParams(needs_layout_passes=False),
        scratch_types=[
            pltpu.VMEM((RPW,), jnp.int32),
            pltpu.VMEM((RPW, C), jnp.float32),
            pltpu.SemaphoreType.DMA,
        ],
    )
    def sc_gather(f_hbm, src_hbm, out_hbm, idx_v, rows_v, sem):
        wid = lax.axis_index("c") * NSUB + lax.axis_index("s")
        base = wid * RPW
        pltpu.sync_copy(src_hbm.at[pl.ds(base, RPW)], idx_v)
        copies = []
        for ch in range(RPW // GCH):
            sl = pl.ds(ch * GCH, GCH)
            copies.append(pltpu.async_copy(
                f_hbm.at[idx_v.at[sl]], rows_v.at[sl], sem))
        for cp in copies:
            cp.wait()
        pltpu.sync_copy(rows_v, out_hbm.at[pl.ds(base, RPW)])

    @functools.partial(
        pl.kernel,
        out_type=jax.ShapeDtypeStruct((N_PAD, C), jnp.float32),
        mesh=mesh,
        compiler_params=pltpu.CompilerParams(needs_layout_passes=False),
        scratch_types=[
            pltpu.VMEM((ZCH, C), jnp.float32),
            pltpu.VMEM((R_CAP, UPS), jnp.int32),
            pltpu.VMEM((UPS,), jnp.int32),
            pltpu.VMEM((UPS, C), jnp.float32),
            pltpu.VMEM((1, UPS, C), jnp.float32),
            pltpu.SemaphoreType.DMA,
        ],
    )
    def sc_scatter(corr_hbm, src2_hbm, udst_hbm, out_hbm,
                   zbuf, idx_v, udst_v, acc, tmp, sem):
        cid = lax.axis_index("c")
        sid = lax.axis_index("s")
        wid = cid * NSUB + sid

        zero16 = jnp.zeros((16,), jnp.float32)

        def _zrow(i, _):
            for v in range(C // 16):
                zbuf[i, pl.ds(v * 16, 16)] = zero16
            return 0

        lax.fori_loop(0, ZCH, _zrow, 0)
        zbase = wid * ZPW
        zcopies = [pltpu.async_copy(
            zbuf, out_hbm.at[pl.ds(zbase + j * ZCH, ZCH)], sem)
            for j in range(ZPW // ZCH)]
        for cp in zcopies:
            cp.wait()

        plsc.subcore_barrier()

        ubase = sid * UPS
        pltpu.sync_copy(udst_hbm.at[cid, pl.ds(ubase, UPS)], udst_v)
        pltpu.sync_copy(src2_hbm.at[cid, :, pl.ds(ubase, UPS)], idx_v)

        def _add_tmp():
            def _arow(i, _):
                for v in range(C // 16):
                    sl = pl.ds(v * 16, 16)
                    acc[i, sl] = acc[i, sl] + tmp[0, i, sl]
                return 0

            lax.fori_loop(0, UPS, _arow, 0)

        c0 = pltpu.async_copy(corr_hbm.at[idx_v.at[0]], acc, sem)
        c1 = pltpu.async_copy(corr_hbm.at[idx_v.at[1]], tmp.at[0], sem)
        c0.wait()
        c1.wait()
        _add_tmp()
        for r in range(2, R_CAP):
            nz = jnp.zeros((16,), jnp.int32)
            for v in range(UPS // 16):
                chunk = idx_v[r, pl.ds(v * 16, 16)]
                nz = jnp.maximum(nz, jnp.where(chunk != ZR, 1, 0))

            @pl.when(jnp.max(nz) > 0)
            def _round():
                pltpu.async_copy(
                    corr_hbm.at[idx_v.at[r]], tmp.at[0], sem).wait()
                _add_tmp()

        pltpu.async_copy(acc, out_hbm.at[udst_v], sem).wait()

    return sc_search, sc_gather, sc_scatter


def _neighbor_tables(coords):
    ci = coords.astype(jnp.int32)
    keys = ci[:, 0] * (G * G) + ci[:, 1] * G + ci[:, 2]
    order = jnp.argsort(keys)
    skeys = keys[order]
    offs = jnp.array([[dx, dy, dz]
                      for dx in (-1, 0, 1)
                      for dy in (-1, 0, 1)
                      for dz in (-1, 0, 1)], jnp.int32)
    nc = ci[None, :, :] + offs[:, None, :]
    inb = jnp.all((nc >= 0) & (nc < G), axis=2)
    nkey = nc[..., 0] * (G * G) + nc[..., 1] * G + nc[..., 2]
    pos = jnp.clip(jnp.searchsorted(skeys, nkey.reshape(-1)).reshape(27, N),
                   0, N - 1)
    found = inb & (skeys[pos] == nkey)
    src = order[pos]
    return found, src


def _group_by_destination(dst_flat):
    P = ROWS
    sort_idx = jnp.argsort(dst_flat).astype(jnp.int32)
    sdst = dst_flat[sort_idx]
    valid_h = sdst < BIG
    flag = jnp.concatenate([jnp.ones((1,), bool),
                            sdst[1:] != sdst[:-1]]) & valid_h
    ar = jnp.arange(P, dtype=jnp.int32)
    runstart = lax.cummax(jnp.where(flag, ar, 0))
    rpos = ar - runstart
    maxrun = jnp.max(jnp.where(valid_h, rpos, 0)) + 1
    overflow = maxrun > R_CAP

    u0 = jnp.sum((flag & (sdst < HALF)).astype(jnp.int32))
    u_total = jnp.sum(flag.astype(jnp.int32))
    overflow |= (u0 > U_PC) | ((u_total - u0) > U_PC)

    ustart = jnp.argsort(jnp.logical_not(flag), stable=True).astype(jnp.int32)

    gaps = (sdst[1:] > sdst[:-1] + 1) & valid_h[:-1] & (sdst[:-1] + 1 < HALF)
    jj = jnp.argmax(gaps)
    hole0 = jnp.where(sdst[0] != 0, 0, sdst[jj] + 1).astype(jnp.int32)

    sc_ix = jnp.arange(NSC, dtype=jnp.int32)[:, None, None]
    u_ix = jnp.arange(U_PC, dtype=jnp.int32)[None, None, :]
    r_ix = jnp.arange(R_CAP, dtype=jnp.int32)[None, :, None]
    guid = sc_ix * u0 + u_ix
    uvalid = jnp.where(sc_ix == 0, u_ix < u0, guid < u_total)
    st = ustart[jnp.minimum(guid, P - 1)]
    q = st + r_ix
    qc = jnp.minimum(q, P - 1)
    entry_ok = uvalid & (q < P) & (sdst[qc] == sdst[st])
    src2_pk = jnp.where(entry_ok, sort_idx[qc], ZR)
    hole = jnp.stack([hole0, jnp.int32(N)])[:, None]
    udst_pk = jnp.where(uvalid[:, 0, :], sdst[st[:, 0, :]], hole)

    return src2_pk, udst_pk, overflow


def kernel(features, coords, W_conv, b_conv, W_lin, b_lin, gamma, beta):
    ci = coords.astype(jnp.int32)
    keys = ci[:, 0] * (G * G) + ci[:, 1] * G + ci[:, 2]
    order = jnp.argsort(keys).astype(jnp.int32)
    skeys = keys[order]
    kpad = jnp.concatenate(
        [keys, jnp.full((N_PAD - N,), PADKEY, jnp.int32)])
    skp = jnp.concatenate(
        [skeys, jnp.full((SKP - N,), BIG - 1, jnp.int32)])
    ordp = jnp.concatenate(
        [order, jnp.zeros((SKP - N,), jnp.int32)])

    xslab = jnp.searchsorted(
        skeys, (jnp.arange(NSLAB + 1, dtype=jnp.int32) << SLAB_SH)
    ).astype(jnp.int32)
    maxslab = jnp.max(xslab[1:] - xslab[:-1])
    xsp = jnp.concatenate(
        [xslab, jnp.full((XSP - NSLAB - 1,), N, jnp.int32)])

    sc_search, sc_gather, sc_scatter = _sc_kernels()
    srcw, dstw, cntw = sc_search(kpad, skp, ordp, xsp)
    src_pad = srcw.reshape(NW, NOFF, CAPW).transpose(1, 0, 2).reshape(ROWS)
    dst_flat = dstw.reshape(NW, NOFF, CAPW).transpose(1, 0, 2).reshape(ROWS)
    cnts = cntw.reshape(NW, 32)[:, :NOFF]
    overflow = jnp.any(cnts >= CAPW) | (maxslab > (1 << BSTEPS))
    valid3 = (jnp.arange(CAPW, dtype=jnp.int32)[None, None, :]
              < cnts.T[:, :, None]).astype(jnp.float32).reshape(NOFF, CAP, 1)

    src2_pk, udst_pk, ovf2 = _group_by_destination(dst_flat)
    overflow |= ovf2

    Wf = _wfold(W_conv, W_lin)
    bf = (b_conv @ W_lin + b_lin).reshape(1, C)
    g2 = gamma.reshape(1, C)
    b2 = beta.reshape(1, C)

    def _fast(_):
        gc = sc_gather(features, src_pad)
        corr = _corr_mm(gc.reshape(NOFF, CAP, C), valid3, Wf)
        cbuf = sc_scatter(corr.reshape(ROWS, C), src2_pk, udst_pk)
        return _main(features, cbuf, Wf, bf, g2, b2)

    def _dense(_):
        found, src = _neighbor_tables(coords)
        gath = features[src] * found[..., None].astype(features.dtype)
        h = jnp.einsum("knc,kco->no", gath, W_conv) + b_conv
        h = h @ W_lin + b_lin
        mu = jnp.mean(h, axis=-1, keepdims=True)
        var = jnp.var(h, axis=-1, keepdims=True)
        return (h - mu) / jnp.sqrt(var + 1e-5) * gamma + beta

    return lax.cond(overflow, _dense, _fast, 0)
